# Initial kernel scaffold; baseline (speedup 1.0000x reference)
import jax
import jax.numpy as jnp
from jax.experimental import pallas as pl


def kernel(x_table, x_column, x_ontology, e_has_column, e_rev_has_column, e_aligns_tab, e_rev_aligns_tab, e_aligns_col, e_rev_aligns_col, e_references, c1_Wl, c1_Wr, c1_b, c2_Wl, c2_Wr, c2_b, c3_Wl, c3_Wr, c3_b):
    raise NotImplementedError("write your pallas kernel here")


# speedup vs baseline: 1.3122x; 1.3122x over previous
import functools

import jax
import jax.numpy as jnp
from jax import lax
from jax.experimental import pallas as pl
from jax.experimental.pallas import tpu as pltpu
from jax.experimental.pallas import tpu_sc as plsc

_D = 128
_B = 256
_TRASH = 128
_CH_COL = 11264
_NCH_COL = 9
_NPAD_COL = _CH_COL * _NCH_COL
_NPAD_SMALL = 10240
_ZROWS = _CH_COL + _TRASH

_NREAL = {'table': 10000, 'column': 100000, 'ontology': 10000}
_NPAD = {'table': _NPAD_SMALL, 'column': _NPAD_COL, 'ontology': _NPAD_SMALL}

_RELS = [
    ('e_has_column', 'table', 'column'),
    ('e_rev_has_column', 'column', 'table'),
    ('e_aligns_tab', 'ontology', 'table'),
    ('e_rev_aligns_tab', 'table', 'ontology'),
    ('e_aligns_col', 'ontology', 'column'),
    ('e_rev_aligns_col', 'column', 'ontology'),
    ('e_references', 'column', 'column'),
]


def _pad_rows(x, n):
    return jnp.pad(x, ((0, n - x.shape[0]), (0, 0)))


def _pad_edges(e, n_src):
    ne = e.shape[1]
    per32 = ((ne + 31) // 32 + _B - 1) // _B * _B
    total = 32 * per32
    pad_n = total - ne
    if pad_n == 0:
        return e
    ar = jnp.arange(pad_n, dtype=jnp.int32)
    src_pad = ar % n_src
    dst_pad = -(1 + ar % 96)
    return jnp.concatenate([e, jnp.stack([src_pad, dst_pad])], axis=1)


def _make_segsum(e_pad, n_pad, ch, n_chunks, partial, with_counts):
    per32 = e_pad // 32
    nb = (per32 // _B) if partial else (2 * per32 // _B)
    n_rounds = 1 if partial else (n_chunks + 1) // 2
    out_rows = 2 * n_pad if partial else n_pad
    stripe = (ch + _TRASH) // 16
    fs = ch // 16
    nj = _B // 128

    out_type = [jax.ShapeDtypeStruct((out_rows, _D), jnp.float32)]
    if with_counts:
        out_type.append(jax.ShapeDtypeStruct((out_rows,), jnp.float32))
    scratch = [
        pltpu.VMEM((_B,), jnp.int32),
        pltpu.VMEM((_B,), jnp.int32),
        pltpu.VMEM((nj, 128), jnp.int32),
        pltpu.VMEM((nj, 128), jnp.int32),
        pltpu.VMEM((_B, _D), jnp.float32),
        pltpu.VMEM((128,), jnp.float32),
        pltpu.VMEM((stripe,), jnp.float32),
        pltpu.SemaphoreType.DMA,
        pltpu.VMEM_SHARED((ch + _TRASH, _D), jnp.float32),
    ]
    if with_counts:
        scratch.append(pltpu.VMEM_SHARED((ch + _TRASH,), jnp.float32))

    mesh = plsc.VectorSubcoreMesh(core_axis_name="c", subcore_axis_name="s",
                                  num_cores=2, num_subcores=16)

    def body(x_ref, e_ref, z2_ref, z1_ref, *rest):
        if with_counts:
            out_ref, cnt_ref = rest[0], rest[1]
            (src1, dst1, sidx, didx, rows, ones_v, cbuf, sem, acc,
             cntacc) = rest[2:]
        else:
            out_ref = rest[0]
            src1, dst1, sidx, didx, rows, ones_v, cbuf, sem, acc = rest[1:]
            cnt_ref = cntacc = None
        c = lax.axis_index("c")
        s = lax.axis_index("s")
        if with_counts:
            for i in range(8):
                ones_v[pl.ds(i * 16, 16)] = jnp.full((16,), 1.0, jnp.float32)

        def round_body(base, out_off, tile_base):
            pltpu.sync_copy(z2_ref.at[pl.ds(s * stripe, stripe)],
                            acc.at[pl.ds(s * stripe, stripe)])
            if with_counts:
                pltpu.sync_copy(z1_ref.at[pl.ds(s * stripe, stripe)], cbuf)
                pltpu.sync_copy(cbuf, cntacc.at[pl.ds(s * stripe, stripe)])
            plsc.subcore_barrier()
            basev = jnp.full((16,), base, jnp.int32)

            def bat(bi, carry):
                e0 = tile_base + bi * _B
                pltpu.sync_copy(e_ref.at[0, pl.ds(e0, _B)], src1)
                pltpu.sync_copy(e_ref.at[1, pl.ds(e0, _B)], dst1)
                for j in range(nj):
                    for k in range(8):
                        o = j * 128 + k * 16
                        d = dst1[pl.ds(o, 16)]
                        loc = d - basev
                        oob = (loc < 0) | (loc >= ch)
                        tr = (d & (_TRASH - 1)) + ch
                        didx[j, pl.ds(k * 16, 16)] = jnp.where(oob, tr, loc)
                        sidx[j, pl.ds(k * 16, 16)] = src1[pl.ds(o, 16)]
                cps = [pltpu.async_copy(x_ref.at[sidx.at[j]],
                                        rows.at[pl.ds(j * 128, 128)], sem)
                       for j in range(nj)]
                for cp in cps:
                    cp.wait()
                for j in range(nj):
                    pltpu.sync_copy(rows.at[pl.ds(j * 128, 128)],
                                    acc.at[didx.at[j]], add=True)
                    if with_counts:
                        pltpu.sync_copy(ones_v, cntacc.at[didx.at[j]],
                                        add=True)
                return carry

            lax.fori_loop(0, nb, bat, 0)
            plsc.subcore_barrier()
            pltpu.sync_copy(acc.at[pl.ds(s * fs, fs)],
                            out_ref.at[pl.ds(out_off + s * fs, fs)])
            if with_counts:
                pltpu.sync_copy(cntacc.at[pl.ds(s * fs, fs)],
                                cbuf.at[pl.ds(0, fs)])
                pltpu.sync_copy(cbuf.at[pl.ds(0, fs)],
                                cnt_ref.at[pl.ds(out_off + s * fs, fs)])
            plsc.subcore_barrier()

        if partial:
            round_body(0, c * n_pad, (s * 2 + c) * per32)
        else:
            for r in range(n_rounds):
                chunk = r * 2 + c
                if r == n_rounds - 1 and n_chunks % 2 == 1:
                    @pl.when(chunk < n_chunks)
                    def _():
                        round_body(chunk * ch, chunk * ch, s * 2 * per32)
                else:
                    round_body(chunk * ch, chunk * ch, s * 2 * per32)

    return pl.kernel(body,
                     out_type=tuple(out_type) if with_counts else out_type[0],
                     mesh=mesh, scratch_types=scratch)


def _layer_matmul(x_pad, seg_list, cnt_list, Wl, Wr, b, rel_ids, relu):
    n_pad = x_pad.shape[0]
    R = 512
    grid = (n_pad // R,)
    nl = Wl.shape[0]
    k_div = float(len(rel_ids))

    in_specs = [pl.BlockSpec((R, _D), lambda i: (i, 0))]
    operands = [x_pad]
    views = []
    for seg, cnt in zip(seg_list, cnt_list):
        nv = seg.shape[0] // n_pad
        views.append(nv)
        for v in range(nv):
            off = v * (n_pad // R)
            in_specs.append(
                pl.BlockSpec((R, _D), lambda i, off=off: (i + off, 0)))
            operands.append(seg)
        for v in range(nv):
            off = v * (n_pad // R)
            in_specs.append(
                pl.BlockSpec((R, 1), lambda i, off=off: (i + off, 0)))
            operands.append(cnt)
    in_specs += [pl.BlockSpec((nl, _D, _D), lambda i: (0, 0, 0)),
                 pl.BlockSpec((nl, _D, _D), lambda i: (0, 0, 0)),
                 pl.BlockSpec((nl, _D), lambda i: (0, 0))]
    operands += [Wl, Wr, b]

    def mmbody(*refs):
        x_ref, rest, out_ref = refs[0], refs[1:-1], refs[-1]
        pos = 0
        per_rel = []
        for nv in views:
            svs = rest[pos:pos + nv]
            pos += nv
            cvs = rest[pos:pos + nv]
            pos += nv
            per_rel.append((svs, cvs))
        Wl_ref, Wr_ref, b_ref = rest[pos], rest[pos + 1], rest[pos + 2]
        Wr_s = Wr_ref[rel_ids[0]]
        b_s = b_ref[rel_ids[0]]
        for i in rel_ids[1:]:
            Wr_s = Wr_s + Wr_ref[i]
            b_s = b_s + b_ref[i]
        acc = jnp.dot(x_ref[...], Wr_s.T,
                      preferred_element_type=jnp.float32) + b_s[None, :]
        for (svs, cvs), i in zip(per_rel, rel_ids):
            seg = svs[0][...]
            cnt = cvs[0][...]
            for v in range(1, len(svs)):
                seg = seg + svs[v][...]
                cnt = cnt + cvs[v][...]
            mean = seg / jnp.maximum(cnt, 1.0)
            acc = acc + jnp.dot(mean, Wl_ref[i].T,
                                preferred_element_type=jnp.float32)
        acc = acc * (1.0 / k_div)
        if relu:
            acc = jnp.maximum(acc, 0.0)
        out_ref[...] = acc

    return pl.pallas_call(
        mmbody, grid=grid, in_specs=in_specs,
        out_specs=pl.BlockSpec((R, _D), lambda i: (i, 0)),
        out_shape=jax.ShapeDtypeStruct((n_pad, _D), jnp.float32),
    )(*operands)


def kernel(x_table, x_column, x_ontology, e_has_column, e_rev_has_column,
           e_aligns_tab, e_rev_aligns_tab, e_aligns_col, e_rev_aligns_col,
           e_references, c1_Wl, c1_Wr, c1_b, c2_Wl, c2_Wr, c2_b, c3_Wl,
           c3_Wr, c3_b):
    x = {'table': _pad_rows(x_table, _NPAD_SMALL),
         'column': _pad_rows(x_column, _NPAD_COL),
         'ontology': _pad_rows(x_ontology, _NPAD_SMALL)}
    raw_edges = {'e_has_column': e_has_column,
                 'e_rev_has_column': e_rev_has_column,
                 'e_aligns_tab': e_aligns_tab,
                 'e_rev_aligns_tab': e_rev_aligns_tab,
                 'e_aligns_col': e_aligns_col,
                 'e_rev_aligns_col': e_rev_aligns_col,
                 'e_references': e_references}
    edges = {}
    for name, s_t, d_t in _RELS:
        edges[name] = _pad_edges(raw_edges[name], _NREAL[s_t])
    zeros2 = jnp.zeros((_ZROWS, _D), jnp.float32)
    zeros1 = jnp.zeros((_ZROWS,), jnp.float32)

    def seg_call(name, s_t, d_t, xs, with_counts):
        partial = d_t != 'column'
        ch = _NPAD[d_t] if partial else _CH_COL
        nch = 2 if partial else _NCH_COL
        kfn = _make_segsum(edges[name].shape[1], _NPAD[d_t], ch, nch,
                           partial, with_counts)
        return kfn(xs, edges[name], zeros2, zeros1)

    segs, cnts = {}, {}
    for name, s_t, d_t in _RELS:
        seg, cnt = seg_call(name, s_t, d_t, x[s_t], True)
        segs[name] = seg
        cnts[name] = cnt.reshape(-1, 1)

    x1 = {
        'table': _layer_matmul(
            x['table'],
            [segs['e_rev_has_column'], segs['e_aligns_tab']],
            [cnts['e_rev_has_column'], cnts['e_aligns_tab']],
            c1_Wl, c1_Wr, c1_b, [1, 2], True),
        'column': _layer_matmul(
            x['column'],
            [segs['e_has_column'], segs['e_aligns_col'], segs['e_references']],
            [cnts['e_has_column'], cnts['e_aligns_col'], cnts['e_references']],
            c1_Wl, c1_Wr, c1_b, [0, 4, 6], True),
        'ontology': _layer_matmul(
            x['ontology'],
            [segs['e_rev_aligns_tab'], segs['e_rev_aligns_col']],
            [cnts['e_rev_aligns_tab'], cnts['e_rev_aligns_col']],
            c1_Wl, c1_Wr, c1_b, [3, 5], True),
    }

    segs2 = {}
    for name, s_t, d_t in _RELS[:4]:
        segs2[name] = seg_call(name, s_t, d_t, x1[s_t], False)
    x2 = {
        'table': _layer_matmul(
            x1['table'],
            [segs2['e_rev_has_column'], segs2['e_aligns_tab']],
            [cnts['e_rev_has_column'], cnts['e_aligns_tab']],
            c2_Wl, c2_Wr, c2_b, [1, 2], True),
        'column': _layer_matmul(
            x1['column'], [segs2['e_has_column']], [cnts['e_has_column']],
            c2_Wl, c2_Wr, c2_b, [0], True),
        'ontology': _layer_matmul(
            x1['ontology'],
            [segs2['e_rev_aligns_tab']], [cnts['e_rev_aligns_tab']],
            c2_Wl, c2_Wr, c2_b, [3], True),
    }

    segs3 = {}
    for name, s_t, d_t in _RELS[:2]:
        segs3[name] = seg_call(name, s_t, d_t, x2[s_t], False)
    out_table = _layer_matmul(
        x2['table'], [segs3['e_rev_has_column']], [cnts['e_rev_has_column']],
        c3_Wl, c3_Wr, c3_b, [1], False)
    out_column = _layer_matmul(
        x2['column'], [segs3['e_has_column']], [cnts['e_has_column']],
        c3_Wl, c3_Wr, c3_b, [0], False)
    return (out_table[:_NREAL['table']], out_column[:_NREAL['column']])
